# Initial kernel scaffold; baseline (speedup 1.0000x reference)
import jax
import jax.numpy as jnp
from jax.experimental import pallas as pl


def kernel(x, edge_index, batch, W1, b1, gamma1, beta1, W2, b2, Wc, bc):
    raise NotImplementedError("write your pallas kernel here")


# speedup vs baseline: 15.3973x; 15.3973x over previous
import functools

import jax
import jax.numpy as jnp
from jax import lax
from jax.experimental import pallas as pl
from jax.experimental.pallas import tpu as pltpu
from jax.experimental.pallas import tpu_sc as plsc

_N = 10000
_E = 320000
_F = 128
_H = 128
_C = 10
_G = 128

_NC = 2
_NS = 16
_CH = 128
_ROWS = _E // _CH
_CPT = 80
_CPL = _ROWS - 31 * _CPT
_NRT = 632
_NRL = _N - 15 * _NRT

_HF = 64
_RB = 400
_NB = _N // _RB


def _copy_node_rows(s, src_at, dst_at):
    @pl.when(s < _NS - 1)
    def _():
        pltpu.sync_copy(src_at(s * _NRT, _NRT), dst_at(s * _NRT, _NRT))

    @pl.when(s == _NS - 1)
    def _():
        pltpu.sync_copy(src_at(15 * _NRT, _NRL), dst_at(15 * _NRT, _NRL))


def _load_chunk_rows(hbm_rows, vmem_rows, wid):
    @pl.when(wid < _NC * _NS - 1)
    def _():
        pltpu.sync_copy(hbm_rows.at[pl.ds(wid * _CPT, _CPT)], vmem_rows)

    @pl.when(wid == _NC * _NS - 1)
    def _():
        pltpu.sync_copy(hbm_rows.at[pl.ds(31 * _CPT, _CPL)],
                        vmem_rows.at[pl.ds(0, _CPL)])


def _deg_part_body(dst_rows, zeros16, ones16, out, didx, ones_v, acc):
    c = lax.axis_index("c")
    s = lax.axis_index("s")
    wid = c * _NS + s
    nch = jnp.where(wid == _NC * _NS - 1, _CPL, _CPT)
    _copy_node_rows(s, lambda o, n: zeros16.at[pl.ds(o, n)],
                    lambda o, n: acc.at[pl.ds(o, n)])
    pltpu.sync_copy(ones16, ones_v)
    _load_chunk_rows(dst_rows, didx, wid)
    plsc.subcore_barrier()

    def body(j, carry):
        pltpu.sync_copy(ones_v, acc.at[didx.at[j]], add=True)
        return carry

    lax.fori_loop(0, nch, body, 0)
    plsc.subcore_barrier()
    _copy_node_rows(s, lambda o, n: acc.at[pl.ds(o, n)],
                    lambda o, n: out.at[c, pl.ds(o, n)])


def _prop_body(hs_lo, hs_hi, src_rows, dst_rows, zerosHf, out_lo, out_hi,
               sidx, didx, bufa, acc):
    c = lax.axis_index("c")
    s = lax.axis_index("s")
    wid = c * _NS + s
    nch = jnp.where(wid == _NC * _NS - 1, _CPL, _CPT)

    _load_chunk_rows(src_rows, sidx, wid)
    _load_chunk_rows(dst_rows, didx, wid)

    for hs, out in ((hs_lo, out_lo), (hs_hi, out_hi)):
        @pl.when(c == 0)
        def _():
            _copy_node_rows(s, lambda o, n: hs.at[pl.ds(o, n)],
                            lambda o, n: acc.at[pl.ds(o, n)])

        @pl.when(c == 1)
        def _():
            _copy_node_rows(s, lambda o, n: zerosHf.at[pl.ds(o, n)],
                            lambda o, n: acc.at[pl.ds(o, n)])

        plsc.subcore_barrier()

        def body(j, carry):
            pltpu.sync_copy(hs.at[sidx.at[j]], bufa)
            pltpu.sync_copy(bufa, acc.at[didx.at[j]], add=True)
            return carry

        lax.fori_loop(0, nch, body, 0)
        plsc.subcore_barrier()
        _copy_node_rows(s, lambda o, n: acc.at[pl.ds(o, n)],
                        lambda o, n: out.at[c, pl.ds(o, n)])


@functools.cache
def _sc_kernels():
    mesh = plsc.VectorSubcoreMesh(core_axis_name="c", subcore_axis_name="s",
                                  num_cores=_NC, num_subcores=_NS)
    deg_part = pl.kernel(
        _deg_part_body,
        compiler_params=pltpu.CompilerParams(use_tc_tiling_on_sc=False),
        out_type=jax.ShapeDtypeStruct((_NC, _N, 16), jnp.float32),
        mesh=mesh,
        scratch_types=[
            pltpu.VMEM((_CPT, _CH), jnp.int32),
            pltpu.VMEM((_CH, 16), jnp.float32),
            pltpu.VMEM_SHARED((_N, 16), jnp.float32),
        ],
    )
    prop = pl.kernel(
        _prop_body,
        compiler_params=pltpu.CompilerParams(use_tc_tiling_on_sc=False),
        out_type=[jax.ShapeDtypeStruct((_NC, _N, _HF), jnp.float32),
                  jax.ShapeDtypeStruct((_NC, _N, _HF), jnp.float32)],
        mesh=mesh,
        scratch_types=[
            pltpu.VMEM((_CPT, _CH), jnp.int32),
            pltpu.VMEM((_CPT, _CH), jnp.int32),
            pltpu.VMEM((_CH, _HF), jnp.float32),
            pltpu.VMEM_SHARED((_N, _HF), jnp.float32),
        ],
    )
    return deg_part, prop


def _dinv_from_parts(dp_ref):
    dp = dp_ref[0] + dp_ref[1]
    deg = jnp.sum(dp, axis=1, keepdims=True) * (1.0 / 16.0) + 1.0
    return lax.rsqrt(deg)


def _k1_body(x_ref, w_ref, dp_ref, lo_ref, hi_ref):
    dinv = _dinv_from_parts(dp_ref)
    hs = jnp.dot(x_ref[...], w_ref[...],
                 preferred_element_type=jnp.float32) * dinv
    lo_ref[...] = hs[:, :_HF]
    hi_ref[...] = hs[:, _HF:]


def _u_from_parts(plo_ref, phi_ref, dinv):
    return jnp.concatenate(
        [plo_ref[0] + plo_ref[1], phi_ref[0] + phi_ref[1]], axis=-1) * dinv


def _k3a_body(plo_ref, phi_ref, dp_ref, b1_ref, g_ref, beta_ref, out_ref):
    i = pl.program_id(0)
    dinv = _dinv_from_parts(dp_ref)
    u = _u_from_parts(plo_ref, phi_ref, dinv) + b1_ref[...]

    @pl.when(i == 0)
    def _():
        out_ref[...] = jnp.zeros_like(out_ref)

    out_ref[0:1, :] += jnp.sum(u, axis=0, keepdims=True)
    out_ref[1:2, :] += jnp.sum(u * u, axis=0, keepdims=True)

    @pl.when(i == _NB - 1)
    def _():
        mu = out_ref[0:1, :] * (1.0 / _N)
        var = out_ref[1:2, :] * (1.0 / _N) - mu * mu
        scale = g_ref[...] * lax.rsqrt(var + 1e-5)
        shift = beta_ref[...] - mu * scale
        out_ref[0:1, :] = scale
        out_ref[1:2, :] = shift


def _k3b_body(plo_ref, phi_ref, dp_ref, b1_ref, ss_ref, w2_ref,
              lo_ref, hi_ref):
    dinv = _dinv_from_parts(dp_ref)
    u = _u_from_parts(plo_ref, phi_ref, dinv) + b1_ref[...]
    v = jnp.maximum(u * ss_ref[0:1, :] + ss_ref[1:2, :], 0.0)
    hs2 = jnp.dot(v, w2_ref[...],
                  preferred_element_type=jnp.float32) * dinv
    lo_ref[...] = hs2[:, :_HF]
    hi_ref[...] = hs2[:, _HF:]


def _k5_body(plo_ref, phi_ref, dp_ref, b2_ref, bat_ref, wc_ref, bc_ref,
             out_ref, g_ref, gacc, cacc):
    i = pl.program_id(0)
    dinv = _dinv_from_parts(dp_ref)
    u2 = _u_from_parts(plo_ref, phi_ref, dinv) + b2_ref[...]
    bid = bat_ref[0]
    iota = lax.broadcasted_iota(jnp.int32, (_G, _RB), 0).astype(jnp.float32)
    oh = jnp.where(iota == bid, 1.0, 0.0)

    @pl.when(i == 0)
    def _():
        gacc[...] = jnp.zeros_like(gacc)
        cacc[...] = jnp.zeros_like(cacc)

    gacc[...] += jnp.dot(oh, u2, preferred_element_type=jnp.float32)
    cacc[...] = cacc[...] + jnp.sum(oh, axis=1, keepdims=True)

    @pl.when(i == _NB - 1)
    def _():
        gg = gacc[...] / jnp.maximum(cacc[...], 1.0)
        g_ref[...] = gg
        logits = jnp.dot(gg, wc_ref[...],
                         preferred_element_type=jnp.float32) + bc_ref[...]
        m = jnp.max(logits, axis=1, keepdims=True)
        lse = jnp.log(jnp.sum(jnp.exp(logits - m), axis=1, keepdims=True)) + m
        out_ref[...] = logits - lse


def _rows(i):
    return (i, 0)


def _rows3(i):
    return (0, i, 0)


def _const2(i):
    return (0, 0)


def kernel(x, edge_index, batch, W1, b1, gamma1, beta1, W2, b2, Wc, bc):
    f32 = jnp.float32
    src_rows = edge_index[0].reshape(_ROWS, _CH)
    dst_rows = edge_index[1].reshape(_ROWS, _CH)
    zeros16 = jnp.zeros((_N, 16), f32)
    zerosHf = jnp.zeros((_N, _HF), f32)
    ones16 = jnp.ones((_CH, 16), f32)
    batch_f = batch.astype(f32).reshape(_NB, 1, _RB)
    b1r = b1.reshape(1, _H)
    g1r = gamma1.reshape(1, _H)
    be1r = beta1.reshape(1, _H)
    b2r = b2.reshape(1, _H)
    bcr = bc.reshape(1, _C)

    deg_part, prop = _sc_kernels()
    degp = deg_part(dst_rows, zeros16, ones16)

    hs1 = pl.pallas_call(
        _k1_body,
        grid=(_NB,),
        in_specs=[pl.BlockSpec((_RB, _F), _rows),
                  pl.BlockSpec((_F, _H), _const2),
                  pl.BlockSpec((_NC, _RB, 16), _rows3)],
        out_specs=[pl.BlockSpec((_RB, _HF), _rows),
                   pl.BlockSpec((_RB, _HF), _rows)],
        out_shape=[jax.ShapeDtypeStruct((_N, _HF), f32),
                   jax.ShapeDtypeStruct((_N, _HF), f32)],
    )(x, W1, degp)

    p1lo, p1hi = prop(hs1[0], hs1[1], src_rows, dst_rows, zerosHf)

    ss = pl.pallas_call(
        _k3a_body,
        grid=(_NB,),
        in_specs=[pl.BlockSpec((_NC, _RB, _HF), _rows3),
                  pl.BlockSpec((_NC, _RB, _HF), _rows3),
                  pl.BlockSpec((_NC, _RB, 16), _rows3),
                  pl.BlockSpec((1, _H), _const2),
                  pl.BlockSpec((1, _H), _const2),
                  pl.BlockSpec((1, _H), _const2)],
        out_specs=pl.BlockSpec((2, _H), _const2),
        out_shape=jax.ShapeDtypeStruct((2, _H), f32),
    )(p1lo, p1hi, degp, b1r, g1r, be1r)

    hs2 = pl.pallas_call(
        _k3b_body,
        grid=(_NB,),
        in_specs=[pl.BlockSpec((_NC, _RB, _HF), _rows3),
                  pl.BlockSpec((_NC, _RB, _HF), _rows3),
                  pl.BlockSpec((_NC, _RB, 16), _rows3),
                  pl.BlockSpec((1, _H), _const2),
                  pl.BlockSpec((2, _H), _const2),
                  pl.BlockSpec((_H, _H), _const2)],
        out_specs=[pl.BlockSpec((_RB, _HF), _rows),
                   pl.BlockSpec((_RB, _HF), _rows)],
        out_shape=[jax.ShapeDtypeStruct((_N, _HF), f32),
                   jax.ShapeDtypeStruct((_N, _HF), f32)],
    )(p1lo, p1hi, degp, b1r, ss, W2)

    p2lo, p2hi = prop(hs2[0], hs2[1], src_rows, dst_rows, zerosHf)

    out, g = pl.pallas_call(
        _k5_body,
        grid=(_NB,),
        in_specs=[pl.BlockSpec((_NC, _RB, _HF), _rows3),
                  pl.BlockSpec((_NC, _RB, _HF), _rows3),
                  pl.BlockSpec((_NC, _RB, 16), _rows3),
                  pl.BlockSpec((1, _H), _const2),
                  pl.BlockSpec((1, 1, _RB), lambda i: (i, 0, 0)),
                  pl.BlockSpec((_H, _C), _const2),
                  pl.BlockSpec((1, _C), _const2)],
        out_specs=[pl.BlockSpec((_G, _C), _const2),
                   pl.BlockSpec((_G, _H), _const2)],
        out_shape=[jax.ShapeDtypeStruct((_G, _C), f32),
                   jax.ShapeDtypeStruct((_G, _H), f32)],
        scratch_shapes=[pltpu.VMEM((_G, _H), f32),
                        pltpu.VMEM((_G, _H), f32)],
    )(p2lo, p2hi, degp, b2r, batch_f, Wc, bcr)

    return (out, g)


# speedup vs baseline: 21.6542x; 1.4064x over previous
import functools

import jax
import jax.numpy as jnp
from jax import lax
from jax.experimental import pallas as pl
from jax.experimental.pallas import tpu as pltpu
from jax.experimental.pallas import tpu_sc as plsc

_N = 10000
_E = 320000
_F = 128
_H = 128
_C = 10
_G = 128

_NC = 2
_NS = 16
_CH = 128
_ROWS = _E // _CH
_CPT = 80
_CPL = _ROWS - 31 * _CPT
_NRT = 632
_NRL = _N - 15 * _NRT

_HF = 64
_RB = 400
_NB = _N // _RB


def _copy_node_rows(s, src_at, dst_at):
    @pl.when(s < _NS - 1)
    def _():
        pltpu.sync_copy(src_at(s * _NRT, _NRT), dst_at(s * _NRT, _NRT))

    @pl.when(s == _NS - 1)
    def _():
        pltpu.sync_copy(src_at(15 * _NRT, _NRL), dst_at(15 * _NRT, _NRL))


def _load_chunk_rows(hbm_rows, vmem_rows, wid):
    @pl.when(wid < _NC * _NS - 1)
    def _():
        pltpu.sync_copy(hbm_rows.at[pl.ds(wid * _CPT, _CPT)], vmem_rows)

    @pl.when(wid == _NC * _NS - 1)
    def _():
        pltpu.sync_copy(hbm_rows.at[pl.ds(31 * _CPT, _CPL)],
                        vmem_rows.at[pl.ds(0, _CPL)])


def _deg_part_body(dst_rows, zeros16, ones16, out, didx, ones_v, acc):
    c = lax.axis_index("c")
    s = lax.axis_index("s")
    wid = c * _NS + s
    nch = jnp.where(wid == _NC * _NS - 1, _CPL, _CPT)
    _copy_node_rows(s, lambda o, n: zeros16.at[pl.ds(o, n)],
                    lambda o, n: acc.at[pl.ds(o, n)])
    pltpu.sync_copy(ones16, ones_v)
    _load_chunk_rows(dst_rows, didx, wid)
    plsc.subcore_barrier()

    def body(j, carry):
        pltpu.sync_copy(ones_v, acc.at[didx.at[j]], add=True)
        return carry

    lax.fori_loop(0, nch, body, 0)
    plsc.subcore_barrier()
    _copy_node_rows(s, lambda o, n: acc.at[pl.ds(o, n)],
                    lambda o, n: out.at[c, pl.ds(o, n)])


def _prop_body(hs_lo, hs_hi, src_rows, dst_rows, zerosHf, out_lo, out_hi,
               sidx, didx, bufa, bufb, sema, semb, acc):
    c = lax.axis_index("c")
    s = lax.axis_index("s")
    wid = c * _NS + s
    nch = jnp.where(wid == _NC * _NS - 1, _CPL, _CPT)

    _load_chunk_rows(src_rows, sidx, wid)
    _load_chunk_rows(dst_rows, didx, wid)

    for hs, out in ((hs_lo, out_lo), (hs_hi, out_hi)):
        @pl.when(c == 0)
        def _():
            _copy_node_rows(s, lambda o, n: hs.at[pl.ds(o, n)],
                            lambda o, n: acc.at[pl.ds(o, n)])

        @pl.when(c == 1)
        def _():
            _copy_node_rows(s, lambda o, n: zerosHf.at[pl.ds(o, n)],
                            lambda o, n: acc.at[pl.ds(o, n)])

        plsc.subcore_barrier()

        pltpu.async_copy(hs.at[sidx.at[0]], bufa, sema)

        def body(i, carry):
            j = 2 * i
            pltpu.make_async_copy(hs.at[sidx.at[j]], bufa, sema).wait()
            pltpu.async_copy(hs.at[sidx.at[j + 1]], bufb, semb)
            pltpu.sync_copy(bufa, acc.at[didx.at[j]], add=True)
            jn = jnp.minimum(j + 2, nch - 1)
            pltpu.async_copy(hs.at[sidx.at[jn]], bufa, sema)
            pltpu.make_async_copy(hs.at[sidx.at[j + 1]], bufb, semb).wait()
            pltpu.sync_copy(bufb, acc.at[didx.at[j + 1]], add=True)
            return carry

        lax.fori_loop(0, nch // 2, body, 0)
        pltpu.make_async_copy(hs.at[sidx.at[0]], bufa, sema).wait()
        plsc.subcore_barrier()
        _copy_node_rows(s, lambda o, n: acc.at[pl.ds(o, n)],
                        lambda o, n: out.at[c, pl.ds(o, n)])


@functools.cache
def _sc_kernels():
    mesh = plsc.VectorSubcoreMesh(core_axis_name="c", subcore_axis_name="s",
                                  num_cores=_NC, num_subcores=_NS)
    deg_part = pl.kernel(
        _deg_part_body,
        compiler_params=pltpu.CompilerParams(use_tc_tiling_on_sc=False),
        out_type=jax.ShapeDtypeStruct((_NC, _N, 16), jnp.float32),
        mesh=mesh,
        scratch_types=[
            pltpu.VMEM((_CPT, _CH), jnp.int32),
            pltpu.VMEM((_CH, 16), jnp.float32),
            pltpu.VMEM_SHARED((_N, 16), jnp.float32),
        ],
    )
    prop = pl.kernel(
        _prop_body,
        compiler_params=pltpu.CompilerParams(use_tc_tiling_on_sc=False),
        out_type=[jax.ShapeDtypeStruct((_NC, _N, _HF), jnp.float32),
                  jax.ShapeDtypeStruct((_NC, _N, _HF), jnp.float32)],
        mesh=mesh,
        scratch_types=[
            pltpu.VMEM((_CPT, _CH), jnp.int32),
            pltpu.VMEM((_CPT, _CH), jnp.int32),
            pltpu.VMEM((_CH, _HF), jnp.float32),
            pltpu.VMEM((_CH, _HF), jnp.float32),
            pltpu.SemaphoreType.DMA,
            pltpu.SemaphoreType.DMA,
            pltpu.VMEM_SHARED((_N, _HF), jnp.float32),
        ],
    )
    return deg_part, prop


def _dinv_from_parts(dp_ref):
    dp = dp_ref[0] + dp_ref[1]
    deg = jnp.sum(dp, axis=1, keepdims=True) * (1.0 / 16.0) + 1.0
    return lax.rsqrt(deg)


def _k1_body(x_ref, w_ref, dp_ref, lo_ref, hi_ref):
    dinv = _dinv_from_parts(dp_ref)
    hs = jnp.dot(x_ref[...], w_ref[...],
                 preferred_element_type=jnp.float32) * dinv
    lo_ref[...] = hs[:, :_HF]
    hi_ref[...] = hs[:, _HF:]


def _u_from_parts(plo_ref, phi_ref, dinv):
    return jnp.concatenate(
        [plo_ref[0] + plo_ref[1], phi_ref[0] + phi_ref[1]], axis=-1) * dinv


def _k3a_body(plo_ref, phi_ref, dp_ref, b1_ref, g_ref, beta_ref, out_ref):
    i = pl.program_id(0)
    dinv = _dinv_from_parts(dp_ref)
    u = _u_from_parts(plo_ref, phi_ref, dinv) + b1_ref[...]

    @pl.when(i == 0)
    def _():
        out_ref[...] = jnp.zeros_like(out_ref)

    out_ref[0:1, :] += jnp.sum(u, axis=0, keepdims=True)
    out_ref[1:2, :] += jnp.sum(u * u, axis=0, keepdims=True)

    @pl.when(i == _NB - 1)
    def _():
        mu = out_ref[0:1, :] * (1.0 / _N)
        var = out_ref[1:2, :] * (1.0 / _N) - mu * mu
        scale = g_ref[...] * lax.rsqrt(var + 1e-5)
        shift = beta_ref[...] - mu * scale
        out_ref[0:1, :] = scale
        out_ref[1:2, :] = shift


def _k3b_body(plo_ref, phi_ref, dp_ref, b1_ref, ss_ref, w2_ref,
              lo_ref, hi_ref):
    dinv = _dinv_from_parts(dp_ref)
    u = _u_from_parts(plo_ref, phi_ref, dinv) + b1_ref[...]
    v = jnp.maximum(u * ss_ref[0:1, :] + ss_ref[1:2, :], 0.0)
    hs2 = jnp.dot(v, w2_ref[...],
                  preferred_element_type=jnp.float32) * dinv
    lo_ref[...] = hs2[:, :_HF]
    hi_ref[...] = hs2[:, _HF:]


def _k5_body(plo_ref, phi_ref, dp_ref, b2_ref, bat_ref, wc_ref, bc_ref,
             out_ref, g_ref, gacc, cacc):
    i = pl.program_id(0)
    dinv = _dinv_from_parts(dp_ref)
    u2 = _u_from_parts(plo_ref, phi_ref, dinv) + b2_ref[...]
    bid = bat_ref[0]
    iota = lax.broadcasted_iota(jnp.int32, (_G, _RB), 0).astype(jnp.float32)
    oh = jnp.where(iota == bid, 1.0, 0.0)

    @pl.when(i == 0)
    def _():
        gacc[...] = jnp.zeros_like(gacc)
        cacc[...] = jnp.zeros_like(cacc)

    gacc[...] += jnp.dot(oh, u2, preferred_element_type=jnp.float32)
    cacc[...] = cacc[...] + jnp.sum(oh, axis=1, keepdims=True)

    @pl.when(i == _NB - 1)
    def _():
        gg = gacc[...] / jnp.maximum(cacc[...], 1.0)
        g_ref[...] = gg
        logits = jnp.dot(gg, wc_ref[...],
                         preferred_element_type=jnp.float32) + bc_ref[...]
        m = jnp.max(logits, axis=1, keepdims=True)
        lse = jnp.log(jnp.sum(jnp.exp(logits - m), axis=1, keepdims=True)) + m
        out_ref[...] = logits - lse


def _rows(i):
    return (i, 0)


def _rows3(i):
    return (0, i, 0)


def _const2(i):
    return (0, 0)


def kernel(x, edge_index, batch, W1, b1, gamma1, beta1, W2, b2, Wc, bc):
    f32 = jnp.float32
    src_rows = edge_index[0].reshape(_ROWS, _CH)
    dst_rows = edge_index[1].reshape(_ROWS, _CH)
    zeros16 = jnp.zeros((_N, 16), f32)
    zerosHf = jnp.zeros((_N, _HF), f32)
    ones16 = jnp.ones((_CH, 16), f32)
    batch_f = batch.astype(f32).reshape(_NB, 1, _RB)
    b1r = b1.reshape(1, _H)
    g1r = gamma1.reshape(1, _H)
    be1r = beta1.reshape(1, _H)
    b2r = b2.reshape(1, _H)
    bcr = bc.reshape(1, _C)

    deg_part, prop = _sc_kernels()
    degp = deg_part(dst_rows, zeros16, ones16)

    hs1 = pl.pallas_call(
        _k1_body,
        grid=(_NB,),
        in_specs=[pl.BlockSpec((_RB, _F), _rows),
                  pl.BlockSpec((_F, _H), _const2),
                  pl.BlockSpec((_NC, _RB, 16), _rows3)],
        out_specs=[pl.BlockSpec((_RB, _HF), _rows),
                   pl.BlockSpec((_RB, _HF), _rows)],
        out_shape=[jax.ShapeDtypeStruct((_N, _HF), f32),
                   jax.ShapeDtypeStruct((_N, _HF), f32)],
    )(x, W1, degp)

    p1lo, p1hi = prop(hs1[0], hs1[1], src_rows, dst_rows, zerosHf)

    ss = pl.pallas_call(
        _k3a_body,
        grid=(_NB,),
        in_specs=[pl.BlockSpec((_NC, _RB, _HF), _rows3),
                  pl.BlockSpec((_NC, _RB, _HF), _rows3),
                  pl.BlockSpec((_NC, _RB, 16), _rows3),
                  pl.BlockSpec((1, _H), _const2),
                  pl.BlockSpec((1, _H), _const2),
                  pl.BlockSpec((1, _H), _const2)],
        out_specs=pl.BlockSpec((2, _H), _const2),
        out_shape=jax.ShapeDtypeStruct((2, _H), f32),
    )(p1lo, p1hi, degp, b1r, g1r, be1r)

    hs2 = pl.pallas_call(
        _k3b_body,
        grid=(_NB,),
        in_specs=[pl.BlockSpec((_NC, _RB, _HF), _rows3),
                  pl.BlockSpec((_NC, _RB, _HF), _rows3),
                  pl.BlockSpec((_NC, _RB, 16), _rows3),
                  pl.BlockSpec((1, _H), _const2),
                  pl.BlockSpec((2, _H), _const2),
                  pl.BlockSpec((_H, _H), _const2)],
        out_specs=[pl.BlockSpec((_RB, _HF), _rows),
                   pl.BlockSpec((_RB, _HF), _rows)],
        out_shape=[jax.ShapeDtypeStruct((_N, _HF), f32),
                   jax.ShapeDtypeStruct((_N, _HF), f32)],
    )(p1lo, p1hi, degp, b1r, ss, W2)

    p2lo, p2hi = prop(hs2[0], hs2[1], src_rows, dst_rows, zerosHf)

    out, g = pl.pallas_call(
        _k5_body,
        grid=(_NB,),
        in_specs=[pl.BlockSpec((_NC, _RB, _HF), _rows3),
                  pl.BlockSpec((_NC, _RB, _HF), _rows3),
                  pl.BlockSpec((_NC, _RB, 16), _rows3),
                  pl.BlockSpec((1, _H), _const2),
                  pl.BlockSpec((1, 1, _RB), lambda i: (i, 0, 0)),
                  pl.BlockSpec((_H, _C), _const2),
                  pl.BlockSpec((1, _C), _const2)],
        out_specs=[pl.BlockSpec((_G, _C), _const2),
                   pl.BlockSpec((_G, _H), _const2)],
        out_shape=[jax.ShapeDtypeStruct((_G, _C), f32),
                   jax.ShapeDtypeStruct((_G, _H), f32)],
        scratch_shapes=[pltpu.VMEM((_G, _H), f32),
                        pltpu.VMEM((_G, _H), f32)],
    )(p2lo, p2hi, degp, b2r, batch_f, Wc, bcr)

    return (out, g)
